# Initial kernel scaffold; baseline (speedup 1.0000x reference)
import jax
import jax.numpy as jnp
from jax.experimental import pallas as pl


def kernel(x, tables, W0, W1, W2):
    raise NotImplementedError("write your pallas kernel here")


# speedup vs baseline: 129.2314x; 129.2314x over previous
import functools

import jax
import jax.numpy as jnp
import numpy as np
from jax import lax
from jax.experimental import pallas as pl
from jax.experimental.pallas import tpu as pltpu
from jax.experimental.pallas import tpu_sc as plsc

N = 262144
L = 16
T = 2 ** 19
MASK = T - 1
BASE_RES = 16
SCALE = 1.5
RESOLUTIONS = [int(np.floor(BASE_RES * (SCALE ** l))) for l in range(L)]
PR1 = np.int32(np.uint32(2654435761).astype(np.int64) - (1 << 32))
PR2 = np.int32(805459861)

NC, NS = 2, 16
NW = NC * NS
PPT = N // NW
P = 1024
NCH = PPT // P
GRP = P // 16
ROWS = 8 * P
DCH = 128
ND = ROWS // DCH

D_HIDDEN = 64
D_OUT = 3
BN = 2048


def _sc_encode_body(xT, tab, enc, xvx, xvy, xvz, idxv, wv, rowsv, encv, sem):
    cid = lax.axis_index("c")
    sid = lax.axis_index("s")
    wid = sid * NC + cid
    iota = lax.iota(jnp.int32, 16)

    def chunk_body(ch, carry):
        base = wid * PPT + ch * P
        pltpu.sync_copy(xT.at[pl.ds(base, P)], xvx)
        pltpu.sync_copy(xT.at[pl.ds(N + base, P)], xvy)
        pltpu.sync_copy(xT.at[pl.ds(2 * N + base, P)], xvz)

        for l in range(L):
            res = RESOLUTIONS[l]
            dense = (res + 1) ** 3 <= T
            lbase = l * T
            resf = np.float32(res)
            rmax = np.int32(res - 1)

            def idx_body(g, c2, dense=dense, lbase=lbase, resf=resf,
                         rmax=rmax, res=res):
                o = g * 16
                px = xvx[pl.ds(o, 16)] * resf
                py = xvy[pl.ds(o, 16)] * resf
                pz = xvz[pl.ds(o, 16)] * resf
                ix = jnp.clip(px.astype(jnp.int32), 0, rmax)
                iy = jnp.clip(py.astype(jnp.int32), 0, rmax)
                iz = jnp.clip(pz.astype(jnp.int32), 0, rmax)
                fx = px - ix.astype(jnp.float32)
                fy = py - iy.astype(jnp.float32)
                fz = pz - iz.astype(jnp.float32)
                one = np.float32(1.0)
                wx = (one - fx, fx)
                wy = (one - fy, fy)
                wz = (one - fz, fz)
                if dense:
                    s = np.int32(res + 1)
                    s2 = np.int32((res + 1) * (res + 1))
                    tx = (ix, ix + 1)
                    ty0 = iy * s
                    ty = (ty0, ty0 + s)
                    tz0 = iz * s2 + np.int32(lbase)
                    tz = (tz0, tz0 + s2)
                else:
                    tx = (ix, ix + 1)
                    hy0 = iy * PR1
                    ty = (hy0, hy0 + PR1)
                    hz0 = iz * PR2
                    tz = (hz0, hz0 + PR2)
                for c in range(8):
                    i, j, k = c >> 2, (c >> 1) & 1, c & 1
                    if dense:
                        idx_c = tx[i] + ty[j] + tz[k]
                    else:
                        h = (tx[i] ^ ty[j]) ^ tz[k]
                        idx_c = (h & np.int32(MASK)) + np.int32(lbase)
                    w_c = (wx[i] * wy[j]) * wz[k]
                    idxv[pl.ds(c * P + o, 16)] = idx_c
                    wv[pl.ds(c * P + o, 16)] = w_c
                return c2

            lax.fori_loop(0, GRP, idx_body, 0, unroll=False)

            def fire(j, c2):
                pltpu.async_copy(
                    tab.at[idxv.at[pl.ds(j * DCH, DCH)]],
                    rowsv.at[pl.ds(j * DCH, DCH)],
                    sem,
                )
                return c2

            lax.fori_loop(0, ND, fire, 0, unroll=False)
            pltpu.make_async_copy(tab.at[pl.ds(0, ROWS)], rowsv, sem).wait()

            def red_body(g, c2, l=l):
                o = g * 16
                acc0 = jnp.zeros((16,), jnp.float32)
                acc1 = jnp.zeros((16,), jnp.float32)
                for c in range(8):
                    v = rowsv[pl.ds(c * P + o, 16)]
                    v0 = lax.bitcast_convert_type(v << 16, jnp.float32)
                    v1 = lax.bitcast_convert_type(v & np.int32(-65536),
                                                  jnp.float32)
                    w = wv[pl.ds(c * P + o, 16)]
                    acc0 = acc0 + w * v0
                    acc1 = acc1 + w * v1
                encv[2 * l, pl.ds(o, 16)] = acc0
                encv[2 * l + 1, pl.ds(o, 16)] = acc1
                return c2

            lax.fori_loop(0, GRP, red_body, 0, unroll=False)

        pltpu.sync_copy(encv, enc.at[:, pl.ds(base, P)])
        return carry

    lax.fori_loop(0, NCH, chunk_body, 0, unroll=False)


_sc_encode = functools.partial(
    pl.kernel,
    out_type=jax.ShapeDtypeStruct((2 * L, N), jnp.float32),
    mesh=plsc.VectorSubcoreMesh(core_axis_name="c", subcore_axis_name="s"),
    scratch_types=[
        pltpu.VMEM((P,), jnp.float32),
        pltpu.VMEM((P,), jnp.float32),
        pltpu.VMEM((P,), jnp.float32),
        pltpu.VMEM((ROWS,), jnp.int32),
        pltpu.VMEM((ROWS,), jnp.float32),
        pltpu.VMEM((ROWS,), jnp.int32),
        pltpu.VMEM((2 * L, P), jnp.float32),
        pltpu.SemaphoreType.DMA,
    ],
)(_sc_encode_body)


def _mlp_body(enc_ref, w0_ref, w1_ref, w2_ref, out_ref):
    e = enc_ref[...]
    h = lax.dot_general(e, w0_ref[...], (((0,), (0,)), ((), ())),
                        preferred_element_type=jnp.float32)
    h = jnp.maximum(h, 0.0)
    h = jnp.dot(h, w1_ref[...], preferred_element_type=jnp.float32)
    h = jnp.maximum(h, 0.0)
    out_ref[...] = jnp.dot(h, w2_ref[...], preferred_element_type=jnp.float32)


def _mlp(enc, W0, W1, W2):
    return pl.pallas_call(
        _mlp_body,
        grid=(N // BN,),
        in_specs=[
            pl.BlockSpec((2 * L, BN), lambda i: (0, i)),
            pl.BlockSpec((2 * L, D_HIDDEN), lambda i: (0, 0)),
            pl.BlockSpec((D_HIDDEN, D_HIDDEN), lambda i: (0, 0)),
            pl.BlockSpec((D_HIDDEN, D_OUT), lambda i: (0, 0)),
        ],
        out_specs=pl.BlockSpec((BN, D_OUT), lambda i: (i, 0)),
        out_shape=jax.ShapeDtypeStruct((N, D_OUT), jnp.float32),
    )(enc, W0, W1, W2)


def kernel(x, tables, W0, W1, W2):
    xT = x.T.reshape(3 * N)
    tab = lax.bitcast_convert_type(
        tables.astype(jnp.bfloat16), jnp.int32).reshape(L * T)
    enc = _sc_encode(xT, tab)
    return _mlp(enc, W0, W1, W2)


# speedup vs baseline: 145.3184x; 1.1245x over previous
import functools

import jax
import jax.numpy as jnp
import numpy as np
from jax import lax
from jax.experimental import pallas as pl
from jax.experimental.pallas import tpu as pltpu
from jax.experimental.pallas import tpu_sc as plsc

N = 262144
L = 16
T = 2 ** 19
MASK = T - 1
BASE_RES = 16
SCALE = 1.5
RESOLUTIONS = [int(np.floor(BASE_RES * (SCALE ** l))) for l in range(L)]
PR1 = np.int32(np.uint32(2654435761).astype(np.int64) - (1 << 32))
PR2 = np.int32(805459861)

NC, NS = 2, 16
NW = NC * NS
PPT = N // NW
P = 1024
NCH = PPT // P
GRP = P // 16
ROWS = 8 * P
DCH = 128
ND = ROWS // DCH

D_HIDDEN = 64
D_OUT = 3
BN = 2048


def _sc_encode_body(xT, tab, enc, xvx, xvy, xvz, idx0, idx1, wv0, wv1,
                    rows0, rows1, encv, sem0, sem1):
    cid = lax.axis_index("c")
    sid = lax.axis_index("s")
    wid = sid * NC + cid
    iota = lax.iota(jnp.int32, 16)
    idxs, wvs, rows, sems = (idx0, idx1), (wv0, wv1), (rows0, rows1), (sem0, sem1)

    def chunk_body(ch, carry):
        base = wid * PPT + ch * P
        pltpu.sync_copy(xT.at[pl.ds(base, P)], xvx)
        pltpu.sync_copy(xT.at[pl.ds(N + base, P)], xvy)
        pltpu.sync_copy(xT.at[pl.ds(2 * N + base, P)], xvz)

        def do_idx(l):
            res = RESOLUTIONS[l]
            dense = (res + 1) ** 3 <= T
            lbase = l * T
            resf = np.float32(res)
            rmax = np.int32(res - 1)
            idxv, wv = idxs[l & 1], wvs[l & 1]

            def idx_body(g, c2, dense=dense, lbase=lbase, resf=resf,
                         rmax=rmax, res=res, idxv=idxv, wv=wv):
                o = g * 16
                px = xvx[pl.ds(o, 16)] * resf
                py = xvy[pl.ds(o, 16)] * resf
                pz = xvz[pl.ds(o, 16)] * resf
                ix = jnp.clip(px.astype(jnp.int32), 0, rmax)
                iy = jnp.clip(py.astype(jnp.int32), 0, rmax)
                iz = jnp.clip(pz.astype(jnp.int32), 0, rmax)
                fx = px - ix.astype(jnp.float32)
                fy = py - iy.astype(jnp.float32)
                fz = pz - iz.astype(jnp.float32)
                one = np.float32(1.0)
                wx = (one - fx, fx)
                wy = (one - fy, fy)
                wz = (one - fz, fz)
                if dense:
                    s = np.int32(res + 1)
                    s2 = np.int32((res + 1) * (res + 1))
                    tx = (ix, ix + 1)
                    ty0 = iy * s
                    ty = (ty0, ty0 + s)
                    tz0 = iz * s2 + np.int32(lbase)
                    tz = (tz0, tz0 + s2)
                else:
                    tx = (ix, ix + 1)
                    hy0 = iy * PR1
                    ty = (hy0, hy0 + PR1)
                    hz0 = iz * PR2
                    tz = (hz0, hz0 + PR2)
                for c in range(8):
                    i, j, k = c >> 2, (c >> 1) & 1, c & 1
                    if dense:
                        idx_c = tx[i] + ty[j] + tz[k]
                    else:
                        h = (tx[i] ^ ty[j]) ^ tz[k]
                        idx_c = (h & np.int32(MASK)) + np.int32(lbase)
                    w_c = (wx[i] * wy[j]) * wz[k]
                    idxv[pl.ds(c * P + o, 16)] = idx_c
                    wv[pl.ds(c * P + o, 16)] = w_c
                return c2

            lax.fori_loop(0, GRP, idx_body, 0, unroll=False)

        def do_fire(l):
            idxv, rowsv, sem = idxs[l & 1], rows[l & 1], sems[l & 1]

            def fire(j, c2, idxv=idxv, rowsv=rowsv, sem=sem):
                pltpu.async_copy(
                    tab.at[idxv.at[pl.ds(j * DCH, DCH)]],
                    rowsv.at[pl.ds(j * DCH, DCH)],
                    sem,
                )
                return c2

            lax.fori_loop(0, ND, fire, 0, unroll=False)

        def do_drain(l):
            pltpu.make_async_copy(tab.at[pl.ds(0, ROWS)], rows[l & 1],
                                  sems[l & 1]).wait()

        def do_reduce(l):
            rowsv, wv = rows[l & 1], wvs[l & 1]

            def red_body(g, c2, l=l, rowsv=rowsv, wv=wv):
                o = g * 16
                acc0 = jnp.zeros((16,), jnp.float32)
                acc1 = jnp.zeros((16,), jnp.float32)
                for c in range(8):
                    v = rowsv[pl.ds(c * P + o, 16)]
                    v0 = lax.bitcast_convert_type(v << 16, jnp.float32)
                    v1 = lax.bitcast_convert_type(v & np.int32(-65536),
                                                  jnp.float32)
                    w = wv[pl.ds(c * P + o, 16)]
                    acc0 = acc0 + w * v0
                    acc1 = acc1 + w * v1
                encv[2 * l, pl.ds(o, 16)] = acc0
                encv[2 * l + 1, pl.ds(o, 16)] = acc1
                return c2

            lax.fori_loop(0, GRP, red_body, 0, unroll=False)

        do_idx(0)
        do_fire(0)
        for l in range(1, L):
            do_idx(l)
            do_drain(l - 1)
            do_fire(l)
            do_reduce(l - 1)
        do_drain(L - 1)
        do_reduce(L - 1)

        pltpu.sync_copy(encv, enc.at[:, pl.ds(base, P)])
        return carry

    lax.fori_loop(0, NCH, chunk_body, 0, unroll=False)


_sc_encode = functools.partial(
    pl.kernel,
    out_type=jax.ShapeDtypeStruct((2 * L, N), jnp.float32),
    mesh=plsc.VectorSubcoreMesh(core_axis_name="c", subcore_axis_name="s"),
    scratch_types=[
        pltpu.VMEM((P,), jnp.float32),
        pltpu.VMEM((P,), jnp.float32),
        pltpu.VMEM((P,), jnp.float32),
        pltpu.VMEM((ROWS,), jnp.int32),
        pltpu.VMEM((ROWS,), jnp.int32),
        pltpu.VMEM((ROWS,), jnp.float32),
        pltpu.VMEM((ROWS,), jnp.float32),
        pltpu.VMEM((ROWS,), jnp.int32),
        pltpu.VMEM((ROWS,), jnp.int32),
        pltpu.VMEM((2 * L, P), jnp.float32),
        pltpu.SemaphoreType.DMA,
        pltpu.SemaphoreType.DMA,
    ],
)(_sc_encode_body)


def _mlp_body(enc_ref, w0_ref, w1_ref, w2_ref, out_ref):
    e = enc_ref[...]
    h = lax.dot_general(e, w0_ref[...], (((0,), (0,)), ((), ())),
                        preferred_element_type=jnp.float32)
    h = jnp.maximum(h, 0.0)
    h = jnp.dot(h, w1_ref[...], preferred_element_type=jnp.float32)
    h = jnp.maximum(h, 0.0)
    out_ref[...] = jnp.dot(h, w2_ref[...], preferred_element_type=jnp.float32)


def _mlp(enc, W0, W1, W2):
    return pl.pallas_call(
        _mlp_body,
        grid=(N // BN,),
        in_specs=[
            pl.BlockSpec((2 * L, BN), lambda i: (0, i)),
            pl.BlockSpec((2 * L, D_HIDDEN), lambda i: (0, 0)),
            pl.BlockSpec((D_HIDDEN, D_HIDDEN), lambda i: (0, 0)),
            pl.BlockSpec((D_HIDDEN, D_OUT), lambda i: (0, 0)),
        ],
        out_specs=pl.BlockSpec((BN, D_OUT), lambda i: (i, 0)),
        out_shape=jax.ShapeDtypeStruct((N, D_OUT), jnp.float32),
    )(enc, W0, W1, W2)


def kernel(x, tables, W0, W1, W2):
    xT = x.T.reshape(3 * N)
    tab = lax.bitcast_convert_type(
        tables.astype(jnp.bfloat16), jnp.int32).reshape(L * T)
    enc = _sc_encode(xT, tab)
    return _mlp(enc, W0, W1, W2)


# speedup vs baseline: 157.1349x; 1.0813x over previous
import functools

import jax
import jax.numpy as jnp
import numpy as np
from jax import lax
from jax.experimental import pallas as pl
from jax.experimental.pallas import tpu as pltpu
from jax.experimental.pallas import tpu_sc as plsc

N = 262144
L = 16
T = 2 ** 19
MASK = T - 1
BASE_RES = 16
SCALE = 1.5
RESOLUTIONS = [int(np.floor(BASE_RES * (SCALE ** l))) for l in range(L)]
PR1 = np.int32(np.uint32(2654435761).astype(np.int64) - (1 << 32))
PR2 = np.int32(805459861)

NC, NS = 2, 16
NW = NC * NS
PPT = N // NW
P = 1024
NCH = PPT // P
GRP = P // 16
ROWS = 8 * P
DCH = 128
ND = ROWS // DCH

D_HIDDEN = 64
D_OUT = 3
BN = 2048


def _sc_encode_body(xT, tab, enc, xvx, xvy, xvz, idx0, idx1, wv0, wv1,
                    rows0, rows1, encv, sem0, sem1):
    cid = lax.axis_index("c")
    sid = lax.axis_index("s")
    wid = sid * NC + cid
    iota = lax.iota(jnp.int32, 16)
    idxs, wvs, rows, sems = (idx0, idx1), (wv0, wv1), (rows0, rows1), (sem0, sem1)

    def chunk_body(ch, carry):
        base = wid * PPT + ch * P

        def do_idx(l):
            res = RESOLUTIONS[l]
            dense = (res + 1) ** 3 <= T
            lbase = l * T
            resf = np.float32(res)
            rmax = np.int32(res - 1)
            idxv, wv = idxs[l & 1], wvs[l & 1]

            def idx_body(g, c2, dense=dense, lbase=lbase, resf=resf,
                         rmax=rmax, res=res, idxv=idxv, wv=wv):
                o = g * 16
                px = xvx[pl.ds(o, 16)] * resf
                py = xvy[pl.ds(o, 16)] * resf
                pz = xvz[pl.ds(o, 16)] * resf
                ix = jnp.clip(px.astype(jnp.int32), 0, rmax)
                iy = jnp.clip(py.astype(jnp.int32), 0, rmax)
                iz = jnp.clip(pz.astype(jnp.int32), 0, rmax)
                fx = px - ix.astype(jnp.float32)
                fy = py - iy.astype(jnp.float32)
                fz = pz - iz.astype(jnp.float32)
                one = np.float32(1.0)
                wx = (one - fx, fx)
                wy = (one - fy, fy)
                wz = (one - fz, fz)
                if dense:
                    s = np.int32(res + 1)
                    s2 = np.int32((res + 1) * (res + 1))
                    tx = (ix, ix + 1)
                    ty0 = iy * s
                    ty = (ty0, ty0 + s)
                    tz0 = iz * s2 + np.int32(lbase)
                    tz = (tz0, tz0 + s2)
                else:
                    tx = (ix, ix + 1)
                    hy0 = iy * PR1
                    ty = (hy0, hy0 + PR1)
                    hz0 = iz * PR2
                    tz = (hz0, hz0 + PR2)
                for c in range(8):
                    i, j, k = c >> 2, (c >> 1) & 1, c & 1
                    if dense:
                        idx_c = tx[i] + ty[j] + tz[k]
                    else:
                        h = (tx[i] ^ ty[j]) ^ tz[k]
                        idx_c = (h & np.int32(MASK)) + np.int32(lbase)
                    w_c = (wx[i] * wy[j]) * wz[k]
                    idxv[pl.ds(c * P + o, 16)] = idx_c
                    wv[pl.ds(c * P + o, 16)] = w_c
                return c2

            lax.fori_loop(0, GRP, idx_body, 0, unroll=False)

        def do_fire(l):
            idxv, rowsv, sem = idxs[l & 1], rows[l & 1], sems[l & 1]

            def fire(j, c2, idxv=idxv, rowsv=rowsv, sem=sem):
                pltpu.async_copy(
                    tab.at[idxv.at[pl.ds(j * DCH, DCH)]],
                    rowsv.at[pl.ds(j * DCH, DCH)],
                    sem,
                )
                return c2

            lax.fori_loop(0, ND, fire, 0, unroll=False)

        def do_drain(l):
            pltpu.make_async_copy(tab.at[pl.ds(0, ROWS)], rows[l & 1],
                                  sems[l & 1]).wait()

        def do_reduce(l):
            rowsv, wv = rows[l & 1], wvs[l & 1]

            def red_body(g, c2, l=l, rowsv=rowsv, wv=wv):
                o = g * 16
                acc0 = jnp.zeros((16,), jnp.float32)
                acc1 = jnp.zeros((16,), jnp.float32)
                for c in range(8):
                    v = rowsv[pl.ds(c * P + o, 16)]
                    v0 = lax.bitcast_convert_type(v << 16, jnp.float32)
                    v1 = lax.bitcast_convert_type(v & np.int32(-65536),
                                                  jnp.float32)
                    w = wv[pl.ds(c * P + o, 16)]
                    acc0 = acc0 + w * v0
                    acc1 = acc1 + w * v1
                encv[2 * l, pl.ds(o, 16)] = acc0
                encv[2 * l + 1, pl.ds(o, 16)] = acc1
                return c2

            lax.fori_loop(0, GRP, red_body, 0, unroll=False)

        @pl.when(ch > 0)
        def _tail():
            do_drain(L - 2)
            do_reduce(L - 2)
            do_drain(L - 1)
            do_reduce(L - 1)
            pltpu.sync_copy(encv, enc.at[:, pl.ds(base - P, P)])

        pltpu.sync_copy(xT.at[pl.ds(base, P)], xvx)
        pltpu.sync_copy(xT.at[pl.ds(N + base, P)], xvy)
        pltpu.sync_copy(xT.at[pl.ds(2 * N + base, P)], xvz)

        do_idx(0)
        do_fire(0)
        do_idx(1)
        do_fire(1)
        for l in range(2, L):
            do_drain(l - 2)
            do_reduce(l - 2)
            do_idx(l)
            do_fire(l)
        return carry

    lax.fori_loop(0, NCH, chunk_body, 0, unroll=False)

    base_last = wid * PPT + (NCH - 1) * P

    def final_reduce(l):
        pltpu.make_async_copy(tab.at[pl.ds(0, ROWS)], rows[l & 1],
                              sems[l & 1]).wait()

        def red_body(g, c2, l=l):
            o = g * 16
            acc0 = jnp.zeros((16,), jnp.float32)
            acc1 = jnp.zeros((16,), jnp.float32)
            for c in range(8):
                v = rows[l & 1][pl.ds(c * P + o, 16)]
                v0 = lax.bitcast_convert_type(v << 16, jnp.float32)
                v1 = lax.bitcast_convert_type(v & np.int32(-65536),
                                              jnp.float32)
                w = wvs[l & 1][pl.ds(c * P + o, 16)]
                acc0 = acc0 + w * v0
                acc1 = acc1 + w * v1
            encv[2 * l, pl.ds(o, 16)] = acc0
            encv[2 * l + 1, pl.ds(o, 16)] = acc1
            return c2

        lax.fori_loop(0, GRP, red_body, 0, unroll=False)

    final_reduce(L - 2)
    final_reduce(L - 1)
    pltpu.sync_copy(encv, enc.at[:, pl.ds(base_last, P)])


_sc_encode = functools.partial(
    pl.kernel,
    out_type=jax.ShapeDtypeStruct((2 * L, N), jnp.float32),
    mesh=plsc.VectorSubcoreMesh(core_axis_name="c", subcore_axis_name="s"),
    scratch_types=[
        pltpu.VMEM((P,), jnp.float32),
        pltpu.VMEM((P,), jnp.float32),
        pltpu.VMEM((P,), jnp.float32),
        pltpu.VMEM((ROWS,), jnp.int32),
        pltpu.VMEM((ROWS,), jnp.int32),
        pltpu.VMEM((ROWS,), jnp.float32),
        pltpu.VMEM((ROWS,), jnp.float32),
        pltpu.VMEM((ROWS,), jnp.int32),
        pltpu.VMEM((ROWS,), jnp.int32),
        pltpu.VMEM((2 * L, P), jnp.float32),
        pltpu.SemaphoreType.DMA,
        pltpu.SemaphoreType.DMA,
    ],
)(_sc_encode_body)


def _mlp_body(enc_ref, w0_ref, w1_ref, w2_ref, out_ref):
    e = enc_ref[...]
    h = lax.dot_general(e, w0_ref[...], (((0,), (0,)), ((), ())),
                        preferred_element_type=jnp.float32)
    h = jnp.maximum(h, 0.0)
    h = jnp.dot(h, w1_ref[...], preferred_element_type=jnp.float32)
    h = jnp.maximum(h, 0.0)
    out_ref[...] = jnp.dot(h, w2_ref[...], preferred_element_type=jnp.float32)


def _mlp(enc, W0, W1, W2):
    return pl.pallas_call(
        _mlp_body,
        grid=(N // BN,),
        in_specs=[
            pl.BlockSpec((2 * L, BN), lambda i: (0, i)),
            pl.BlockSpec((2 * L, D_HIDDEN), lambda i: (0, 0)),
            pl.BlockSpec((D_HIDDEN, D_HIDDEN), lambda i: (0, 0)),
            pl.BlockSpec((D_HIDDEN, D_OUT), lambda i: (0, 0)),
        ],
        out_specs=pl.BlockSpec((BN, D_OUT), lambda i: (i, 0)),
        out_shape=jax.ShapeDtypeStruct((N, D_OUT), jnp.float32),
    )(enc, W0, W1, W2)


def kernel(x, tables, W0, W1, W2):
    xT = x.T.reshape(3 * N)
    tab = lax.bitcast_convert_type(
        tables.astype(jnp.bfloat16), jnp.int32).reshape(L * T)
    enc = _sc_encode(xT, tab)
    return _mlp(enc, W0, W1, W2)


# speedup vs baseline: 199.5709x; 1.2701x over previous
import functools

import jax
import jax.numpy as jnp
import numpy as np
from jax import lax
from jax.experimental import pallas as pl
from jax.experimental.pallas import tpu as pltpu
from jax.experimental.pallas import tpu_sc as plsc

N = 262144
L = 16
T = 2 ** 19
MASK = T - 1
BASE_RES = 16
SCALE = 1.5
RESOLUTIONS = [int(np.floor(BASE_RES * (SCALE ** l))) for l in range(L)]
PR1 = np.int32(np.uint32(2654435761).astype(np.int64) - (1 << 32))
PR2 = np.int32(805459861)

NC, NS = 2, 16
NW = NC * NS
PPT = N // NW
P = 1024
NCH = PPT // P
GRP = P // 16
ROWS = 8 * P
DCH = 128
ND = ROWS // DCH

D_HIDDEN = 64
D_OUT = 3
BN = 2048

STAGED = [0, 1, 2, 4]
_used = {l: ((RESOLUTIONS[l] + 1) ** 3 if (RESOLUTIONS[l] + 1) ** 3 <= T
             else T) for l in STAGED}
_pad = {l: (_used[l] + 127) // 128 * 128 for l in STAGED}
SOFF = {}
_off = 0
for _l in STAGED:
    SOFF[_l] = _off
    _off += _pad[_l]
SP_ROWS = _off
LEVORD = [3, 0, 5, 1, 6, 2, 7, 4, 8, 9, 10, 11, 12, 13, 14, 15]


def _sc_encode_body(xT, tab, enc, xvx, xvy, xvz, idx0, idx1, wv0, wv1,
                    rows0, rows1, encv, spm, sem0, sem1):
    cid = lax.axis_index("c")
    sid = lax.axis_index("s")
    wid = sid * NC + cid
    iota = lax.iota(jnp.int32, 16)
    idxs, wvs, rows, sems = (idx0, idx1), (wv0, wv1), (rows0, rows1), (sem0, sem1)

    for _sl in STAGED:
        _splen = _pad[_sl] // NS
        _hbase = _sl * T + sid * _splen
        _sbase = SOFF[_sl] + sid * _splen
        for _ho in range(0, _splen, ROWS):
            _hl = min(ROWS, _splen - _ho)
            pltpu.sync_copy(tab.at[pl.ds(_hbase + _ho, _hl)],
                            rows0.at[pl.ds(0, _hl)])
            pltpu.sync_copy(rows0.at[pl.ds(0, _hl)],
                            spm.at[pl.ds(_sbase + _ho, _hl)])

    plsc.subcore_barrier()

    def chunk_body(ch, carry):
        base = wid * PPT + ch * P

        def do_idx(l, par):
            res = RESOLUTIONS[l]
            dense = (res + 1) ** 3 <= T
            lbase = SOFF[l] if l in SOFF else l * T
            resf = np.float32(res)
            rmax = np.int32(res - 1)
            idxv, wv = idxs[par], wvs[par]

            def idx_body(g, c2, dense=dense, lbase=lbase, resf=resf,
                         rmax=rmax, res=res, idxv=idxv, wv=wv):
                o = g * 16
                px = xvx[pl.ds(o, 16)] * resf
                py = xvy[pl.ds(o, 16)] * resf
                pz = xvz[pl.ds(o, 16)] * resf
                ix = jnp.clip(px.astype(jnp.int32), 0, rmax)
                iy = jnp.clip(py.astype(jnp.int32), 0, rmax)
                iz = jnp.clip(pz.astype(jnp.int32), 0, rmax)
                fx = px - ix.astype(jnp.float32)
                fy = py - iy.astype(jnp.float32)
                fz = pz - iz.astype(jnp.float32)
                one = np.float32(1.0)
                wx = (one - fx, fx)
                wy = (one - fy, fy)
                wz = (one - fz, fz)
                if dense:
                    s = np.int32(res + 1)
                    s2 = np.int32((res + 1) * (res + 1))
                    tx = (ix, ix + 1)
                    ty0 = iy * s
                    ty = (ty0, ty0 + s)
                    tz0 = iz * s2 + np.int32(lbase)
                    tz = (tz0, tz0 + s2)
                else:
                    tx = (ix, ix + 1)
                    hy0 = iy * PR1
                    ty = (hy0, hy0 + PR1)
                    hz0 = iz * PR2
                    tz = (hz0, hz0 + PR2)
                for c in range(8):
                    i, j, k = c >> 2, (c >> 1) & 1, c & 1
                    if dense:
                        idx_c = tx[i] + ty[j] + tz[k]
                    else:
                        h = (tx[i] ^ ty[j]) ^ tz[k]
                        idx_c = (h & np.int32(MASK)) + np.int32(lbase)
                    w_c = (wx[i] * wy[j]) * wz[k]
                    idxv[pl.ds(c * P + o, 16)] = idx_c
                    wv[pl.ds(c * P + o, 16)] = w_c
                return c2

            lax.fori_loop(0, GRP, idx_body, 0, unroll=False)

        def do_fire(l, par):
            src = spm if l in SOFF else tab
            idxv, rowsv, sem = idxs[par], rows[par], sems[par]

            def fire(j, c2, src=src, idxv=idxv, rowsv=rowsv, sem=sem):
                pltpu.async_copy(
                    src.at[idxv.at[pl.ds(j * DCH, DCH)]],
                    rowsv.at[pl.ds(j * DCH, DCH)],
                    sem,
                )
                return c2

            lax.fori_loop(0, ND, fire, 0, unroll=False)

        def do_drain(par):
            pltpu.make_async_copy(tab.at[pl.ds(0, ROWS)], rows[par],
                                  sems[par]).wait()

        def do_reduce(l, par):
            rowsv, wv = rows[par], wvs[par]

            def red_body(g, c2, l=l, rowsv=rowsv, wv=wv):
                o = g * 16
                acc0 = jnp.zeros((16,), jnp.float32)
                acc1 = jnp.zeros((16,), jnp.float32)
                for c in range(8):
                    v = rowsv[pl.ds(c * P + o, 16)]
                    v0 = lax.bitcast_convert_type(v << 16, jnp.float32)
                    v1 = lax.bitcast_convert_type(v & np.int32(-65536),
                                                  jnp.float32)
                    w = wv[pl.ds(c * P + o, 16)]
                    acc0 = acc0 + w * v0
                    acc1 = acc1 + w * v1
                encv[2 * l, pl.ds(o, 16)] = acc0
                encv[2 * l + 1, pl.ds(o, 16)] = acc1
                return c2

            lax.fori_loop(0, GRP, red_body, 0, unroll=False)

        @pl.when(ch > 0)
        def _tail():
            do_drain(0)
            do_reduce(LEVORD[L - 2], 0)
            do_drain(1)
            do_reduce(LEVORD[L - 1], 1)
            pltpu.sync_copy(encv, enc.at[:, pl.ds(base - P, P)])

        pltpu.sync_copy(xT.at[pl.ds(base, P)], xvx)
        pltpu.sync_copy(xT.at[pl.ds(N + base, P)], xvy)
        pltpu.sync_copy(xT.at[pl.ds(2 * N + base, P)], xvz)

        do_idx(LEVORD[0], 0)
        do_fire(LEVORD[0], 0)
        do_idx(LEVORD[1], 1)
        do_fire(LEVORD[1], 1)
        for pos in range(2, L):
            par = pos & 1
            do_drain(par)
            do_reduce(LEVORD[pos - 2], par)
            do_idx(LEVORD[pos], par)
            do_fire(LEVORD[pos], par)
        return carry

    lax.fori_loop(0, NCH, chunk_body, 0, unroll=False)

    base_last = wid * PPT + (NCH - 1) * P

    def final_reduce(l, par):
        pltpu.make_async_copy(tab.at[pl.ds(0, ROWS)], rows[par],
                              sems[par]).wait()

        def red_body(g, c2, l=l, par=par):
            o = g * 16
            acc0 = jnp.zeros((16,), jnp.float32)
            acc1 = jnp.zeros((16,), jnp.float32)
            for c in range(8):
                v = rows[par][pl.ds(c * P + o, 16)]
                v0 = lax.bitcast_convert_type(v << 16, jnp.float32)
                v1 = lax.bitcast_convert_type(v & np.int32(-65536),
                                              jnp.float32)
                w = wvs[par][pl.ds(c * P + o, 16)]
                acc0 = acc0 + w * v0
                acc1 = acc1 + w * v1
            encv[2 * l, pl.ds(o, 16)] = acc0
            encv[2 * l + 1, pl.ds(o, 16)] = acc1
            return c2

        lax.fori_loop(0, GRP, red_body, 0, unroll=False)

    final_reduce(LEVORD[L - 2], 0)
    final_reduce(LEVORD[L - 1], 1)
    pltpu.sync_copy(encv, enc.at[:, pl.ds(base_last, P)])


_sc_encode = functools.partial(
    pl.kernel,
    out_type=jax.ShapeDtypeStruct((2 * L, N), jnp.float32),
    mesh=plsc.VectorSubcoreMesh(core_axis_name="c", subcore_axis_name="s"),
    scratch_types=[
        pltpu.VMEM((P,), jnp.float32),
        pltpu.VMEM((P,), jnp.float32),
        pltpu.VMEM((P,), jnp.float32),
        pltpu.VMEM((ROWS,), jnp.int32),
        pltpu.VMEM((ROWS,), jnp.int32),
        pltpu.VMEM((ROWS,), jnp.float32),
        pltpu.VMEM((ROWS,), jnp.float32),
        pltpu.VMEM((ROWS,), jnp.int32),
        pltpu.VMEM((ROWS,), jnp.int32),
        pltpu.VMEM((2 * L, P), jnp.float32),
        pltpu.VMEM_SHARED((SP_ROWS,), jnp.int32),
        pltpu.SemaphoreType.DMA,
        pltpu.SemaphoreType.DMA,
    ],
)(_sc_encode_body)


def _mlp_body(enc_ref, w0_ref, w1_ref, w2_ref, out_ref):
    e = enc_ref[...]
    h = lax.dot_general(e, w0_ref[...], (((0,), (0,)), ((), ())),
                        preferred_element_type=jnp.float32)
    h = jnp.maximum(h, 0.0)
    h = jnp.dot(h, w1_ref[...], preferred_element_type=jnp.float32)
    h = jnp.maximum(h, 0.0)
    out_ref[...] = jnp.dot(h, w2_ref[...], preferred_element_type=jnp.float32)


def _mlp(enc, W0, W1, W2):
    return pl.pallas_call(
        _mlp_body,
        grid=(N // BN,),
        in_specs=[
            pl.BlockSpec((2 * L, BN), lambda i: (0, i)),
            pl.BlockSpec((2 * L, D_HIDDEN), lambda i: (0, 0)),
            pl.BlockSpec((D_HIDDEN, D_HIDDEN), lambda i: (0, 0)),
            pl.BlockSpec((D_HIDDEN, D_OUT), lambda i: (0, 0)),
        ],
        out_specs=pl.BlockSpec((BN, D_OUT), lambda i: (i, 0)),
        out_shape=jax.ShapeDtypeStruct((N, D_OUT), jnp.float32),
    )(enc, W0, W1, W2)


def kernel(x, tables, W0, W1, W2):
    xT = x.T.reshape(3 * N)
    tab = lax.bitcast_convert_type(
        tables.astype(jnp.bfloat16), jnp.int32).reshape(L * T)
    enc = _sc_encode(xT, tab)
    return _mlp(enc, W0, W1, W2)


# speedup vs baseline: 207.5653x; 1.0401x over previous
import functools

import jax
import jax.numpy as jnp
import numpy as np
from jax import lax
from jax.experimental import pallas as pl
from jax.experimental.pallas import tpu as pltpu
from jax.experimental.pallas import tpu_sc as plsc

N = 262144
L = 16
T = 2 ** 19
MASK = T - 1
BASE_RES = 16
SCALE = 1.5
RESOLUTIONS = [int(np.floor(BASE_RES * (SCALE ** l))) for l in range(L)]
PR1 = np.int32(np.uint32(2654435761).astype(np.int64) - (1 << 32))
PR2 = np.int32(805459861)

NC, NS = 2, 16
NW = NC * NS
PPT = N // NW
P = 1024
NCH = PPT // P
GRP = P // 16
ROWS = 8 * P
DCH = 128
ND = ROWS // DCH

D_HIDDEN = 64
D_OUT = 3
BN = 8192

STAGED = [0, 1, 2, 4]
_used = {l: ((RESOLUTIONS[l] + 1) ** 3 if (RESOLUTIONS[l] + 1) ** 3 <= T
             else T) for l in STAGED}
_pad = {l: (_used[l] + 127) // 128 * 128 for l in STAGED}
SOFF = {}
_off = 0
for _l in STAGED:
    SOFF[_l] = _off
    _off += _pad[_l]
SP_ROWS = _off
LEVORD = [3, 0, 5, 1, 6, 2, 7, 4, 8, 9, 10, 11, 12, 13, 14, 15]


def _sc_encode_body(xT, tab, enc, xvx, xvy, xvz, idx0, idx1, wv0, wv1,
                    rows0, rows1, encv, spm, sem0, sem1):
    cid = lax.axis_index("c")
    sid = lax.axis_index("s")
    wid = sid * NC + cid
    iota = lax.iota(jnp.int32, 16)
    idxs, wvs, rows, sems = (idx0, idx1), (wv0, wv1), (rows0, rows1), (sem0, sem1)

    for _sl in STAGED:
        _splen = _pad[_sl] // NS
        _hbase = _sl * T + sid * _splen
        _sbase = SOFF[_sl] + sid * _splen
        for _ho in range(0, _splen, ROWS):
            _hl = min(ROWS, _splen - _ho)
            pltpu.sync_copy(tab.at[pl.ds(_hbase + _ho, _hl)],
                            rows0.at[pl.ds(0, _hl)])
            pltpu.sync_copy(rows0.at[pl.ds(0, _hl)],
                            spm.at[pl.ds(_sbase + _ho, _hl)])

    plsc.subcore_barrier()

    def chunk_body(ch, carry):
        base = wid * PPT + ch * P

        def do_idx(l, par):
            res = RESOLUTIONS[l]
            dense = (res + 1) ** 3 <= T
            lbase = SOFF[l] if l in SOFF else l * T
            resf = np.float32(res)
            rmax = np.int32(res - 1)
            idxv, wv = idxs[par], wvs[par]

            def idx_body(g, c2, dense=dense, lbase=lbase, resf=resf,
                         rmax=rmax, res=res, idxv=idxv, wv=wv):
                o = g * 16
                px = xvx[pl.ds(o, 16)] * resf
                py = xvy[pl.ds(o, 16)] * resf
                pz = xvz[pl.ds(o, 16)] * resf
                ix = jnp.clip(px.astype(jnp.int32), 0, rmax)
                iy = jnp.clip(py.astype(jnp.int32), 0, rmax)
                iz = jnp.clip(pz.astype(jnp.int32), 0, rmax)
                fx = px - ix.astype(jnp.float32)
                fy = py - iy.astype(jnp.float32)
                fz = pz - iz.astype(jnp.float32)
                one = np.float32(1.0)
                wx = (one - fx, fx)
                wy = (one - fy, fy)
                wz = (one - fz, fz)
                if dense:
                    s = np.int32(res + 1)
                    s2 = np.int32((res + 1) * (res + 1))
                    tx = (ix, ix + 1)
                    ty0 = iy * s
                    ty = (ty0, ty0 + s)
                    tz0 = iz * s2 + np.int32(lbase)
                    tz = (tz0, tz0 + s2)
                else:
                    tx = (ix, ix + 1)
                    hy0 = iy * PR1
                    ty = (hy0, hy0 + PR1)
                    hz0 = iz * PR2
                    tz = (hz0, hz0 + PR2)
                for c in range(8):
                    i, j, k = c >> 2, (c >> 1) & 1, c & 1
                    if dense:
                        idx_c = tx[i] + ty[j] + tz[k]
                    else:
                        h = (tx[i] ^ ty[j]) ^ tz[k]
                        idx_c = (h & np.int32(MASK)) + np.int32(lbase)
                    w_c = (wx[i] * wy[j]) * wz[k]
                    idxv[pl.ds(c * P + o, 16)] = idx_c
                    wv[pl.ds(c * P + o, 16)] = w_c
                return c2

            lax.fori_loop(0, GRP, idx_body, 0, unroll=False)

        def do_fire(l, par):
            src = spm if l in SOFF else tab
            idxv, rowsv, sem = idxs[par], rows[par], sems[par]

            def fire(j, c2, src=src, idxv=idxv, rowsv=rowsv, sem=sem):
                pltpu.async_copy(
                    src.at[idxv.at[pl.ds(j * DCH, DCH)]],
                    rowsv.at[pl.ds(j * DCH, DCH)],
                    sem,
                )
                return c2

            lax.fori_loop(0, ND, fire, 0, unroll=False)

        def do_drain(par):
            pltpu.make_async_copy(tab.at[pl.ds(0, ROWS)], rows[par],
                                  sems[par]).wait()

        def do_reduce(l, par):
            rowsv, wv = rows[par], wvs[par]

            def red_body(g, c2, l=l, rowsv=rowsv, wv=wv):
                o = g * 16
                acc0 = jnp.zeros((16,), jnp.float32)
                acc1 = jnp.zeros((16,), jnp.float32)
                for c in range(8):
                    v = rowsv[pl.ds(c * P + o, 16)]
                    v0 = lax.bitcast_convert_type(v << 16, jnp.float32)
                    v1 = lax.bitcast_convert_type(v & np.int32(-65536),
                                                  jnp.float32)
                    w = wv[pl.ds(c * P + o, 16)]
                    acc0 = acc0 + w * v0
                    acc1 = acc1 + w * v1
                encv[2 * l, pl.ds(o, 16)] = acc0
                encv[2 * l + 1, pl.ds(o, 16)] = acc1
                return c2

            lax.fori_loop(0, GRP, red_body, 0, unroll=False)

        @pl.when(ch > 0)
        def _tail():
            do_drain(0)
            do_reduce(LEVORD[L - 2], 0)
            do_drain(1)
            do_reduce(LEVORD[L - 1], 1)
            pltpu.sync_copy(encv, enc.at[:, pl.ds(base - P, P)])

        pltpu.sync_copy(xT.at[pl.ds(base, P)], xvx)
        pltpu.sync_copy(xT.at[pl.ds(N + base, P)], xvy)
        pltpu.sync_copy(xT.at[pl.ds(2 * N + base, P)], xvz)

        do_idx(LEVORD[0], 0)
        do_fire(LEVORD[0], 0)
        do_idx(LEVORD[1], 1)
        do_fire(LEVORD[1], 1)
        for pos in range(2, L):
            par = pos & 1
            do_drain(par)
            do_reduce(LEVORD[pos - 2], par)
            do_idx(LEVORD[pos], par)
            do_fire(LEVORD[pos], par)
        return carry

    lax.fori_loop(0, NCH, chunk_body, 0, unroll=False)

    base_last = wid * PPT + (NCH - 1) * P

    def final_reduce(l, par):
        pltpu.make_async_copy(tab.at[pl.ds(0, ROWS)], rows[par],
                              sems[par]).wait()

        def red_body(g, c2, l=l, par=par):
            o = g * 16
            acc0 = jnp.zeros((16,), jnp.float32)
            acc1 = jnp.zeros((16,), jnp.float32)
            for c in range(8):
                v = rows[par][pl.ds(c * P + o, 16)]
                v0 = lax.bitcast_convert_type(v << 16, jnp.float32)
                v1 = lax.bitcast_convert_type(v & np.int32(-65536),
                                              jnp.float32)
                w = wvs[par][pl.ds(c * P + o, 16)]
                acc0 = acc0 + w * v0
                acc1 = acc1 + w * v1
            encv[2 * l, pl.ds(o, 16)] = acc0
            encv[2 * l + 1, pl.ds(o, 16)] = acc1
            return c2

        lax.fori_loop(0, GRP, red_body, 0, unroll=False)

    final_reduce(LEVORD[L - 2], 0)
    final_reduce(LEVORD[L - 1], 1)
    pltpu.sync_copy(encv, enc.at[:, pl.ds(base_last, P)])


_sc_encode = functools.partial(
    pl.kernel,
    out_type=jax.ShapeDtypeStruct((2 * L, N), jnp.float32),
    mesh=plsc.VectorSubcoreMesh(core_axis_name="c", subcore_axis_name="s"),
    scratch_types=[
        pltpu.VMEM((P,), jnp.float32),
        pltpu.VMEM((P,), jnp.float32),
        pltpu.VMEM((P,), jnp.float32),
        pltpu.VMEM((ROWS,), jnp.int32),
        pltpu.VMEM((ROWS,), jnp.int32),
        pltpu.VMEM((ROWS,), jnp.float32),
        pltpu.VMEM((ROWS,), jnp.float32),
        pltpu.VMEM((ROWS,), jnp.int32),
        pltpu.VMEM((ROWS,), jnp.int32),
        pltpu.VMEM((2 * L, P), jnp.float32),
        pltpu.VMEM_SHARED((SP_ROWS,), jnp.int32),
        pltpu.SemaphoreType.DMA,
        pltpu.SemaphoreType.DMA,
    ],
)(_sc_encode_body)


def _mlp_body(enc_ref, w0_ref, w1_ref, w2_ref, out_ref):
    e = enc_ref[...]
    h = lax.dot_general(e, w0_ref[...], (((0,), (0,)), ((), ())),
                        preferred_element_type=jnp.float32)
    h = jnp.maximum(h, 0.0)
    h = jnp.dot(h, w1_ref[...], preferred_element_type=jnp.float32)
    h = jnp.maximum(h, 0.0)
    out_ref[...] = jnp.dot(h, w2_ref[...], preferred_element_type=jnp.float32)


def _mlp(enc, W0, W1, W2):
    return pl.pallas_call(
        _mlp_body,
        grid=(N // BN,),
        in_specs=[
            pl.BlockSpec((2 * L, BN), lambda i: (0, i)),
            pl.BlockSpec((2 * L, D_HIDDEN), lambda i: (0, 0)),
            pl.BlockSpec((D_HIDDEN, D_HIDDEN), lambda i: (0, 0)),
            pl.BlockSpec((D_HIDDEN, D_OUT), lambda i: (0, 0)),
        ],
        out_specs=pl.BlockSpec((BN, D_OUT), lambda i: (i, 0)),
        out_shape=jax.ShapeDtypeStruct((N, D_OUT), jnp.float32),
    )(enc, W0, W1, W2)


def kernel(x, tables, W0, W1, W2):
    xT = x.T.reshape(3 * N)
    tab = lax.bitcast_convert_type(
        tables.astype(jnp.bfloat16), jnp.int32).reshape(L * T)
    enc = _sc_encode(xT, tab)
    return _mlp(enc, W0, W1, W2)


# speedup vs baseline: 208.0918x; 1.0025x over previous
import functools

import jax
import jax.numpy as jnp
import numpy as np
from jax import lax
from jax.experimental import pallas as pl
from jax.experimental.pallas import tpu as pltpu
from jax.experimental.pallas import tpu_sc as plsc

N = 262144
L = 16
T = 2 ** 19
MASK = T - 1
BASE_RES = 16
SCALE = 1.5
RESOLUTIONS = [int(np.floor(BASE_RES * (SCALE ** l))) for l in range(L)]
PR1 = np.int32(np.uint32(2654435761).astype(np.int64) - (1 << 32))
PR2 = np.int32(805459861)

NC, NS = 2, 16
NW = NC * NS
PPT = N // NW
P = 1024
NCH = PPT // P
GRP = P // 16
ROWS = 8 * P
DCH = 128
ND = ROWS // DCH

D_HIDDEN = 64
D_OUT = 3
BN = 16384

STAGED = [0, 1, 2, 4]
_used = {l: ((RESOLUTIONS[l] + 1) ** 3 if (RESOLUTIONS[l] + 1) ** 3 <= T
             else T) for l in STAGED}
_pad = {l: (_used[l] + 127) // 128 * 128 for l in STAGED}
SOFF = {}
_off = 0
for _l in STAGED:
    SOFF[_l] = _off
    _off += _pad[_l]
SP_ROWS = _off
LEVORD = [3, 0, 5, 1, 6, 2, 7, 4, 8, 9, 10, 11, 12, 13, 14, 15]


def _sc_encode_body(xT, tab, enc, xvx, xvy, xvz, idx0, idx1, wv0, wv1,
                    rows0, rows1, encv, spm, sem0, sem1):
    cid = lax.axis_index("c")
    sid = lax.axis_index("s")
    wid = sid * NC + cid
    iota = lax.iota(jnp.int32, 16)
    idxs, wvs, rows, sems = (idx0, idx1), (wv0, wv1), (rows0, rows1), (sem0, sem1)

    for _sl in STAGED:
        _splen = _pad[_sl] // NS
        _hbase = _sl * T + sid * _splen
        _sbase = SOFF[_sl] + sid * _splen
        for _ho in range(0, _splen, ROWS):
            _hl = min(ROWS, _splen - _ho)
            pltpu.sync_copy(tab.at[pl.ds(_hbase + _ho, _hl)],
                            rows0.at[pl.ds(0, _hl)])
            pltpu.sync_copy(rows0.at[pl.ds(0, _hl)],
                            spm.at[pl.ds(_sbase + _ho, _hl)])

    plsc.subcore_barrier()

    def chunk_body(ch, carry):
        base = wid * PPT + ch * P

        def do_idx(l, par):
            res = RESOLUTIONS[l]
            dense = (res + 1) ** 3 <= T
            lbase = SOFF[l] if l in SOFF else l * T
            resf = np.float32(res)
            rmax = np.int32(res - 1)
            idxv, wv = idxs[par], wvs[par]

            def idx_body(g, c2, dense=dense, lbase=lbase, resf=resf,
                         rmax=rmax, res=res, idxv=idxv, wv=wv):
                o = g * 16
                px = xvx[pl.ds(o, 16)] * resf
                py = xvy[pl.ds(o, 16)] * resf
                pz = xvz[pl.ds(o, 16)] * resf
                ix = jnp.clip(px.astype(jnp.int32), 0, rmax)
                iy = jnp.clip(py.astype(jnp.int32), 0, rmax)
                iz = jnp.clip(pz.astype(jnp.int32), 0, rmax)
                fx = px - ix.astype(jnp.float32)
                fy = py - iy.astype(jnp.float32)
                fz = pz - iz.astype(jnp.float32)
                one = np.float32(1.0)
                wx = (one - fx, fx)
                wy = (one - fy, fy)
                wz = (one - fz, fz)
                if dense:
                    s = np.int32(res + 1)
                    s2 = np.int32((res + 1) * (res + 1))
                    tx = (ix, ix + 1)
                    ty0 = iy * s
                    ty = (ty0, ty0 + s)
                    tz0 = iz * s2 + np.int32(lbase)
                    tz = (tz0, tz0 + s2)
                else:
                    tx = (ix, ix + 1)
                    hy0 = iy * PR1
                    ty = (hy0, hy0 + PR1)
                    hz0 = iz * PR2
                    tz = (hz0, hz0 + PR2)
                for c in range(8):
                    i, j, k = c >> 2, (c >> 1) & 1, c & 1
                    if dense:
                        idx_c = tx[i] + ty[j] + tz[k]
                    else:
                        h = (tx[i] ^ ty[j]) ^ tz[k]
                        idx_c = (h & np.int32(MASK)) + np.int32(lbase)
                    w_c = (wx[i] * wy[j]) * wz[k]
                    idxv[pl.ds(c * P + o, 16)] = idx_c
                    wv[pl.ds(c * P + o, 16)] = w_c
                return c2

            lax.fori_loop(0, GRP, idx_body, 0, unroll=False)

        def do_fire(l, par):
            src = spm if l in SOFF else tab
            idxv, rowsv, sem = idxs[par], rows[par], sems[par]

            def fire(j, c2, src=src, idxv=idxv, rowsv=rowsv, sem=sem):
                pltpu.async_copy(
                    src.at[idxv.at[pl.ds(j * DCH, DCH)]],
                    rowsv.at[pl.ds(j * DCH, DCH)],
                    sem,
                )
                return c2

            lax.fori_loop(0, ND, fire, 0, unroll=False)

        def do_drain(par):
            pltpu.make_async_copy(tab.at[pl.ds(0, ROWS)], rows[par],
                                  sems[par]).wait()

        def do_reduce(l, par):
            rowsv, wv = rows[par], wvs[par]

            def red_body(g, c2, l=l, rowsv=rowsv, wv=wv):
                o = g * 16
                acc0 = jnp.zeros((16,), jnp.float32)
                acc1 = jnp.zeros((16,), jnp.float32)
                for c in range(8):
                    v = rowsv[pl.ds(c * P + o, 16)]
                    v0 = lax.bitcast_convert_type(v << 16, jnp.float32)
                    v1 = lax.bitcast_convert_type(v & np.int32(-65536),
                                                  jnp.float32)
                    w = wv[pl.ds(c * P + o, 16)]
                    acc0 = acc0 + w * v0
                    acc1 = acc1 + w * v1
                encv[2 * l, pl.ds(o, 16)] = acc0
                encv[2 * l + 1, pl.ds(o, 16)] = acc1
                return c2

            lax.fori_loop(0, GRP, red_body, 0, unroll=False)

        @pl.when(ch > 0)
        def _tail():
            do_drain(0)
            do_reduce(LEVORD[L - 2], 0)
            do_drain(1)
            do_reduce(LEVORD[L - 1], 1)
            pltpu.sync_copy(encv, enc.at[:, pl.ds(base - P, P)])

        pltpu.sync_copy(xT.at[pl.ds(base, P)], xvx)
        pltpu.sync_copy(xT.at[pl.ds(N + base, P)], xvy)
        pltpu.sync_copy(xT.at[pl.ds(2 * N + base, P)], xvz)

        do_idx(LEVORD[0], 0)
        do_fire(LEVORD[0], 0)
        do_idx(LEVORD[1], 1)
        do_fire(LEVORD[1], 1)
        for pos in range(2, L):
            par = pos & 1
            do_drain(par)
            do_reduce(LEVORD[pos - 2], par)
            do_idx(LEVORD[pos], par)
            do_fire(LEVORD[pos], par)
        return carry

    lax.fori_loop(0, NCH, chunk_body, 0, unroll=False)

    base_last = wid * PPT + (NCH - 1) * P

    def final_reduce(l, par):
        pltpu.make_async_copy(tab.at[pl.ds(0, ROWS)], rows[par],
                              sems[par]).wait()

        def red_body(g, c2, l=l, par=par):
            o = g * 16
            acc0 = jnp.zeros((16,), jnp.float32)
            acc1 = jnp.zeros((16,), jnp.float32)
            for c in range(8):
                v = rows[par][pl.ds(c * P + o, 16)]
                v0 = lax.bitcast_convert_type(v << 16, jnp.float32)
                v1 = lax.bitcast_convert_type(v & np.int32(-65536),
                                              jnp.float32)
                w = wvs[par][pl.ds(c * P + o, 16)]
                acc0 = acc0 + w * v0
                acc1 = acc1 + w * v1
            encv[2 * l, pl.ds(o, 16)] = acc0
            encv[2 * l + 1, pl.ds(o, 16)] = acc1
            return c2

        lax.fori_loop(0, GRP, red_body, 0, unroll=False)

    final_reduce(LEVORD[L - 2], 0)
    final_reduce(LEVORD[L - 1], 1)
    pltpu.sync_copy(encv, enc.at[:, pl.ds(base_last, P)])


_sc_encode = functools.partial(
    pl.kernel,
    out_type=jax.ShapeDtypeStruct((2 * L, N), jnp.float32),
    mesh=plsc.VectorSubcoreMesh(core_axis_name="c", subcore_axis_name="s"),
    scratch_types=[
        pltpu.VMEM((P,), jnp.float32),
        pltpu.VMEM((P,), jnp.float32),
        pltpu.VMEM((P,), jnp.float32),
        pltpu.VMEM((ROWS,), jnp.int32),
        pltpu.VMEM((ROWS,), jnp.int32),
        pltpu.VMEM((ROWS,), jnp.float32),
        pltpu.VMEM((ROWS,), jnp.float32),
        pltpu.VMEM((ROWS,), jnp.int32),
        pltpu.VMEM((ROWS,), jnp.int32),
        pltpu.VMEM((2 * L, P), jnp.float32),
        pltpu.VMEM_SHARED((SP_ROWS,), jnp.int32),
        pltpu.SemaphoreType.DMA,
        pltpu.SemaphoreType.DMA,
    ],
)(_sc_encode_body)


def _mlp_body(enc_ref, w0_ref, w1_ref, w2_ref, out_ref):
    e = enc_ref[...]
    h = lax.dot_general(e, w0_ref[...], (((0,), (0,)), ((), ())),
                        preferred_element_type=jnp.float32)
    h = jnp.maximum(h, 0.0)
    h = jnp.dot(h, w1_ref[...], preferred_element_type=jnp.float32)
    h = jnp.maximum(h, 0.0)
    out_ref[...] = jnp.dot(h, w2_ref[...], preferred_element_type=jnp.float32)


def _mlp(enc, W0, W1, W2):
    return pl.pallas_call(
        _mlp_body,
        grid=(N // BN,),
        in_specs=[
            pl.BlockSpec((2 * L, BN), lambda i: (0, i)),
            pl.BlockSpec((2 * L, D_HIDDEN), lambda i: (0, 0)),
            pl.BlockSpec((D_HIDDEN, D_HIDDEN), lambda i: (0, 0)),
            pl.BlockSpec((D_HIDDEN, D_OUT), lambda i: (0, 0)),
        ],
        out_specs=pl.BlockSpec((BN, D_OUT), lambda i: (i, 0)),
        out_shape=jax.ShapeDtypeStruct((N, D_OUT), jnp.float32),
    )(enc, W0, W1, W2)


def kernel(x, tables, W0, W1, W2):
    xT = x.T.reshape(3 * N)
    tab = lax.bitcast_convert_type(
        tables.astype(jnp.bfloat16), jnp.int32).reshape(L * T)
    enc = _sc_encode(xT, tab)
    return _mlp(enc, W0, W1, W2)
